# baseline (speedup 1.0000x reference)
import functools

import jax
import jax.numpy as jnp
from jax import lax
from jax.experimental import pallas as pl
from jax.experimental.pallas import tpu as pltpu
from jax.experimental.pallas import tpu_sc as plsc

S = 50000
D = 128
NC, NS = 2, 16
NW = NC * NS
NB = 4
C = 120
SPAN_A, SPAN_B = 1568, 1480
NWA = 30
SIZES_A = [120] * 13 + [8]
SIZES_B = [120] * 12 + [40]


def _span_pipeline(base, sizes, x_hbm, sub_hbm, idx_hbm, out_hbm,
                   idx_all, rows, ibuf, spm, slab, sem_i, sem_s, sem_g,
                   sem_a, sem_o):
    K = len(sizes)
    offs = [sum(sizes[:j]) for j in range(K)]
    total = sum(sizes)

    def L(j):
        b = j % NB
        pltpu.async_copy(sub_hbm.at[pl.ds(base + offs[j], sizes[j])],
                         spm.at[pl.ds(slab + b * C, sizes[j])], sem_s[b])

    def G(j):
        b = j % NB
        pltpu.async_copy(x_hbm.at[idx_all.at[pl.ds(offs[j], sizes[j])]],
                         rows[b].at[pl.ds(0, sizes[j])], sem_g[b])

    def A(j):
        b = j % NB
        pltpu.make_async_copy(sub_hbm.at[pl.ds(0, sizes[j])],
                              spm.at[pl.ds(slab, sizes[j])], sem_s[b]).wait()
        pltpu.make_async_copy(x_hbm.at[idx_all.at[pl.ds(offs[j], sizes[j])]],
                              rows[b].at[pl.ds(0, sizes[j])], sem_g[b]).wait()
        pltpu.async_copy(rows[b], spm.at[ibuf[b]], sem_a[b], add=True)

    def St(j):
        b = j % NB
        pltpu.make_async_copy(rows[b], spm.at[ibuf[b]], sem_a[b]).wait()
        pltpu.async_copy(spm.at[pl.ds(slab + b * C, sizes[j])],
                         out_hbm.at[pl.ds(base + offs[j], sizes[j])], sem_o[b])

    def Dr(j):
        b = j % NB
        pltpu.make_async_copy(spm.at[pl.ds(slab, sizes[j])],
                              out_hbm.at[pl.ds(0, sizes[j])], sem_o[b]).wait()

    pltpu.async_copy(idx_hbm.at[pl.ds(base, total)],
                     idx_all.at[pl.ds(0, total)], sem_i)
    L(0)
    L(1)
    pltpu.make_async_copy(idx_hbm.at[pl.ds(0, total)],
                          idx_all.at[pl.ds(0, total)], sem_i).wait()
    G(0)
    G(1)
    drained = -1
    for j in range(K):
        if j + 2 < K:
            if j - 2 >= 0:
                Dr(j - 2)
                drained = j - 2
            L(j + 2)
            G(j + 2)
        if j - 1 >= 0:
            St(j - 1)
        A(j)
    St(K - 1)
    for j in range(drained + 1, K):
        Dr(j)


def _sc_body(x_hbm, sub_hbm, idx_hbm, out_hbm, idx_all, spm, *sc):
    s = lax.axis_index("s")
    wid = s * NC + lax.axis_index("c")
    rows = sc[:NB]
    ibuf = sc[NB:2 * NB]
    si = sc[2 * NB]
    sem_s = sc[2 * NB + 1:3 * NB + 1]
    sem_g = sc[3 * NB + 1:4 * NB + 1]
    sem_a = sc[4 * NB + 1:5 * NB + 1]
    sem_o = sc[5 * NB + 1:6 * NB + 1]

    slab = s * (NB * C)
    lane = lax.iota(jnp.int32, 16)
    for b in range(NB):
        for k in range(C // 16):
            ibuf[b][pl.ds(k * 16, 16)] = lane + (slab + b * C + k * 16)

    args = (x_hbm, sub_hbm, idx_hbm, out_hbm, idx_all, rows, ibuf, spm,
            slab, si, sem_s, sem_g, sem_a, sem_o)

    @pl.when(wid < NWA)
    def _():
        _span_pipeline(wid * SPAN_A, SIZES_A, *args)

    @pl.when(wid >= NWA)
    def _():
        _span_pipeline(NWA * SPAN_A + (wid - NWA) * SPAN_B, SIZES_B, *args)


def kernel(x, sub_representations, new_nodes):
    idx = new_nodes.reshape(-1)

    mesh = plsc.VectorSubcoreMesh(core_axis_name="c", subcore_axis_name="s")
    run = functools.partial(
        pl.kernel,
        mesh=mesh,
        out_type=jax.ShapeDtypeStruct((S, D), jnp.float32),
        scratch_types=(
            [pltpu.VMEM((SPAN_A,), jnp.int32),
             pltpu.VMEM_SHARED((NS * NB * C, D), jnp.float32)]
            + [pltpu.VMEM((C, D), jnp.float32) for _ in range(NB)]
            + [pltpu.VMEM((C,), jnp.int32) for _ in range(NB)]
            + [pltpu.SemaphoreType.DMA for _ in range(1 + 4 * NB)]
        ),
    )(_sc_body)
    return run(x, sub_representations, idx)


# speedup vs baseline: 1.0099x; 1.0099x over previous
import functools

import jax
import jax.numpy as jnp
from jax import lax
from jax.experimental import pallas as pl
from jax.experimental.pallas import tpu as pltpu
from jax.experimental.pallas import tpu_sc as plsc

S = 50000
D = 128
NC, NS = 2, 16
NW = NC * NS
NB = 3
CMAX = 336
SPAN_A, SPAN_B = 1568, 1480
NWA = 30
SIZES_A = [336] * 4 + [224]
SIZES_B = [336] * 4 + [136]


def _span_pipeline(base, sizes, x_hbm, sub_hbm, idx_hbm, out_hbm,
                   idx_all, rows, sem_i, sem_s, sem_g, sem_o):
    K = len(sizes)
    offs = [sum(sizes[:j]) for j in range(K)]
    total = sum(sizes)

    def idx_desc():
        return pltpu.make_async_copy(
            idx_hbm.at[pl.ds(0, total)], idx_all.at[pl.ds(0, total)], sem_i)

    def L(j):
        b = j % NB
        pltpu.async_copy(sub_hbm.at[pl.ds(base + offs[j], sizes[j])],
                         rows[b].at[pl.ds(0, sizes[j])], sem_s[b])

    def G(j):
        b = j % NB
        pltpu.make_async_copy(sub_hbm.at[pl.ds(0, sizes[j])],
                              rows[b].at[pl.ds(0, sizes[j])], sem_s[b]).wait()
        pltpu.async_copy(x_hbm.at[idx_all.at[pl.ds(offs[j], sizes[j])]],
                         rows[b].at[pl.ds(0, sizes[j])], sem_g[b], add=True)

    def W(j):
        b = j % NB
        pltpu.make_async_copy(x_hbm.at[idx_all.at[pl.ds(offs[j], sizes[j])]],
                              rows[b].at[pl.ds(0, sizes[j])], sem_g[b]).wait()
        pltpu.async_copy(rows[b].at[pl.ds(0, sizes[j])],
                         out_hbm.at[pl.ds(base + offs[j], sizes[j])], sem_o[b])

    def Dr(j):
        b = j % NB
        pltpu.make_async_copy(rows[b].at[pl.ds(0, sizes[j])],
                              out_hbm.at[pl.ds(0, sizes[j])], sem_o[b]).wait()

    pltpu.async_copy(idx_hbm.at[pl.ds(base, total)],
                     idx_all.at[pl.ds(0, total)], sem_i)
    L(0)
    if K > 1:
        L(1)
    idx_desc().wait()
    G(0)
    drained = -1
    for j in range(K):
        if j + 2 < K:
            if j + 2 - NB >= 0:
                Dr(j + 2 - NB)
                drained = j + 2 - NB
            L(j + 2)
        W(j)
        if j + 1 < K:
            G(j + 1)
    for j in range(drained + 1, K):
        Dr(j)


def _sc_body(x_hbm, sub_hbm, idx_hbm, out_hbm, idx_all, *sc):
    wid = lax.axis_index("s") * NC + lax.axis_index("c")
    rows = sc[:NB]
    si = sc[NB]
    sem_s = sc[NB + 1:2 * NB + 1]
    sem_g = sc[2 * NB + 1:3 * NB + 1]
    sem_o = sc[3 * NB + 1:4 * NB + 1]
    args = (x_hbm, sub_hbm, idx_hbm, out_hbm,
            idx_all, rows, si, sem_s, sem_g, sem_o)

    @pl.when(wid < NWA)
    def _():
        _span_pipeline(wid * SPAN_A, SIZES_A, *args)

    @pl.when(wid >= NWA)
    def _():
        _span_pipeline(NWA * SPAN_A + (wid - NWA) * SPAN_B, SIZES_B, *args)


def kernel(x, sub_representations, new_nodes):
    idx = new_nodes.reshape(-1)

    mesh = plsc.VectorSubcoreMesh(core_axis_name="c", subcore_axis_name="s")
    run = functools.partial(
        pl.kernel,
        mesh=mesh,
        out_type=jax.ShapeDtypeStruct((S, D), jnp.float32),
        scratch_types=(
            [pltpu.VMEM((SPAN_A,), jnp.int32)]
            + [pltpu.VMEM((CMAX, D), jnp.float32) for _ in range(NB)]
            + [pltpu.SemaphoreType.DMA for _ in range(1 + 3 * NB)]
        ),
    )(_sc_body)
    return run(x, sub_representations, idx)


# speedup vs baseline: 1.0263x; 1.0163x over previous
import functools

import jax
import jax.numpy as jnp
from jax import lax
from jax.experimental import pallas as pl
from jax.experimental.pallas import tpu as pltpu
from jax.experimental.pallas import tpu_sc as plsc

S = 50000
D = 128
NC, NS = 2, 16
NW = NC * NS
NB = 4
CMAX = 248
SPAN_A, SPAN_B = 1568, 1480
NWA = 30
SIZES_A = [248] * 6 + [80]
SIZES_B = [248] * 5 + [240]


def _span_pipeline(base, sizes, x_hbm, sub_hbm, idx_hbm, out_hbm,
                   idx_all, rows, sem_i, sem_s, sem_g, sem_o):
    K = len(sizes)
    offs = [sum(sizes[:j]) for j in range(K)]
    total = sum(sizes)

    def idx_desc():
        return pltpu.make_async_copy(
            idx_hbm.at[pl.ds(0, total)], idx_all.at[pl.ds(0, total)], sem_i)

    def L(j):
        b = j % NB
        pltpu.async_copy(sub_hbm.at[pl.ds(base + offs[j], sizes[j])],
                         rows[b].at[pl.ds(0, sizes[j])], sem_s[b])

    def G(j):
        b = j % NB
        pltpu.make_async_copy(sub_hbm.at[pl.ds(0, sizes[j])],
                              rows[b].at[pl.ds(0, sizes[j])], sem_s[b]).wait()
        pltpu.async_copy(x_hbm.at[idx_all.at[pl.ds(offs[j], sizes[j])]],
                         rows[b].at[pl.ds(0, sizes[j])], sem_g[b], add=True)

    def W(j):
        b = j % NB
        pltpu.make_async_copy(x_hbm.at[idx_all.at[pl.ds(offs[j], sizes[j])]],
                              rows[b].at[pl.ds(0, sizes[j])], sem_g[b]).wait()
        pltpu.async_copy(rows[b].at[pl.ds(0, sizes[j])],
                         out_hbm.at[pl.ds(base + offs[j], sizes[j])], sem_o[b])

    def Dr(j):
        b = j % NB
        pltpu.make_async_copy(rows[b].at[pl.ds(0, sizes[j])],
                              out_hbm.at[pl.ds(0, sizes[j])], sem_o[b]).wait()

    pltpu.async_copy(idx_hbm.at[pl.ds(base, total)],
                     idx_all.at[pl.ds(0, total)], sem_i)
    for j in range(min(NB - 1, K)):
        L(j)
    idx_desc().wait()
    for j in range(min(NB - 1, K)):
        G(j)
    for j in range(K):
        W(j)
        if j - 1 >= 0:
            Dr(j - 1)
        if j + NB - 1 < K:
            L(j + NB - 1)
            G(j + NB - 1)
    Dr(K - 1)


def _sc_body(x_hbm, sub_hbm, idx_hbm, out_hbm, idx_all, *sc):
    wid = lax.axis_index("s") * NC + lax.axis_index("c")
    rows = sc[:NB]
    si = sc[NB]
    sem_s = sc[NB + 1:2 * NB + 1]
    sem_g = sc[2 * NB + 1:3 * NB + 1]
    sem_o = sc[3 * NB + 1:4 * NB + 1]
    args = (x_hbm, sub_hbm, idx_hbm, out_hbm,
            idx_all, rows, si, sem_s, sem_g, sem_o)

    @pl.when(wid < NWA)
    def _():
        _span_pipeline(wid * SPAN_A, SIZES_A, *args)

    @pl.when(wid >= NWA)
    def _():
        _span_pipeline(NWA * SPAN_A + (wid - NWA) * SPAN_B, SIZES_B, *args)


def kernel(x, sub_representations, new_nodes):
    idx = new_nodes.reshape(-1)

    mesh = plsc.VectorSubcoreMesh(core_axis_name="c", subcore_axis_name="s")
    run = functools.partial(
        pl.kernel,
        mesh=mesh,
        out_type=jax.ShapeDtypeStruct((S, D), jnp.float32),
        scratch_types=(
            [pltpu.VMEM((SPAN_A,), jnp.int32)]
            + [pltpu.VMEM((CMAX, D), jnp.float32) for _ in range(NB)]
            + [pltpu.SemaphoreType.DMA for _ in range(1 + 3 * NB)]
        ),
    )(_sc_body)
    return run(x, sub_representations, idx)
